# baseline (speedup 1.0000x reference)
import functools

import jax
import jax.numpy as jnp
from jax import lax
from jax.experimental import pallas as pl
from jax.experimental.pallas import tpu as pltpu
from jax.experimental.pallas import tpu_sc as plsc

B, N = 128, 8192
L = 16
NC, NS = 2, 16
NW = NC * NS
ROWS_PER_W = B // NW
NWIN = 4
CW = N // NWIN
WCHUNKS = CW // L
CWW = CW // 4


def _sc_masked_cumsum(x, mwords):
    mesh = plsc.VectorSubcoreMesh(core_axis_name="c", subcore_axis_name="s")

    @functools.partial(
        pl.kernel,
        mesh=mesh,
        out_type=jax.ShapeDtypeStruct((B, N), jnp.float32),
        compiler_params=pltpu.CompilerParams(needs_layout_passes=False),
        scratch_types=[
            pltpu.VMEM((2, ROWS_PER_W, CW), jnp.float32),
            pltpu.VMEM((2, ROWS_PER_W, CWW), jnp.int32),
            pltpu.VMEM((2, ROWS_PER_W, CW), jnp.float32),
            pltpu.SemaphoreType.DMA,
            pltpu.SemaphoreType.DMA,
            pltpu.SemaphoreType.DMA,
            pltpu.SemaphoreType.DMA,
        ],
    )
    def k(x_hbm, m_hbm, out_hbm, xw, mw, ow, sin0, sin1, sout0, sout1):
        wid = lax.axis_index("s") * NC + lax.axis_index("c")
        base = wid * ROWS_PER_W
        sin = (sin0, sin1)
        sout = (sout0, sout1)

        def start_in(w):
            b = w % 2
            hx = pltpu.async_copy(
                x_hbm.at[pl.ds(base, ROWS_PER_W), pl.ds(w * CW, CW)],
                xw.at[b], sin[b])
            hm = pltpu.async_copy(
                m_hbm.at[pl.ds(base, ROWS_PER_W), pl.ds(w * CWW, CWW)],
                mw.at[b], sin[b])
            return (hx, hm)

        pending_in = {0: start_in(0)}
        pending_out = {}
        carries = (jnp.float32(0.0),) * ROWS_PER_W
        zero = jnp.zeros((L,), jnp.float32)
        lanes = lax.iota(jnp.int32, L)
        bmask = jnp.int32(1) << (8 * (lanes % 4))
        qidx = lanes // 4
        for w in range(NWIN):
            b = w % 2
            for h in pending_in.pop(w):
                h.wait()
            if w + 1 < NWIN:
                pending_in[w + 1] = start_in(w + 1)
            if w - 2 in pending_out:
                pending_out.pop(w - 2).wait()

            def body(i, cs, b=b):
                off = i * L
                new = []
                for r in range(ROWS_PER_W):
                    g = plsc.load_gather(
                        mw, [jnp.full((L,), b, jnp.int32),
                             jnp.full((L,), r, jnp.int32), i * 4 + qidx])
                    v = jnp.where((g & bmask) != 0,
                                  xw[b, r, pl.ds(off, L)], zero)
                    s = jnp.cumsum(v) + cs[r]
                    ow[b, r, pl.ds(off, L)] = s
                    new.append(s[L - 1])
                return tuple(new)

            carries = lax.fori_loop(0, WCHUNKS, body, carries)
            pending_out[w] = pltpu.async_copy(
                ow.at[b],
                out_hbm.at[pl.ds(base, ROWS_PER_W), pl.ds(w * CW, CW)],
                sout[b])
        for w in sorted(pending_out):
            pending_out.pop(w).wait()

    return k(x, mwords)


def kernel(x, mask):
    mwords = lax.bitcast_convert_type(
        mask.astype(jnp.uint8).reshape(B, N // 4, 4), jnp.int32)
    return _sc_masked_cumsum(x, mwords)


# speedup vs baseline: 3.8659x; 3.8659x over previous
import jax
import jax.numpy as jnp
from jax.experimental import pallas as pl
from jax.experimental.pallas import tpu as pltpu

B, N = 128, 8192
CB = 1024
NBLK = N // CB


def _tc_body(x_ref, m_ref, o_ref, carry_ref):
    i = pl.program_id(0)

    @pl.when(i == 0)
    def _():
        carry_ref[...] = jnp.zeros_like(carry_ref)

    s = x_ref[...] * m_ref[...].astype(jnp.float32)
    sh = 1
    while sh < CB:
        s = s + jnp.pad(s[:, :-sh], ((0, 0), (sh, 0)))
        sh *= 2
    o_ref[...] = s + carry_ref[...]
    carry_ref[...] = carry_ref[...] + jnp.broadcast_to(s[:, CB - 1:CB], (B, CB))


def kernel(x, mask):
    return pl.pallas_call(
        _tc_body,
        grid=(NBLK,),
        in_specs=[
            pl.BlockSpec((B, CB), lambda i: (0, i)),
            pl.BlockSpec((B, CB), lambda i: (0, i)),
        ],
        out_specs=pl.BlockSpec((B, CB), lambda i: (0, i)),
        out_shape=jax.ShapeDtypeStruct((B, N), jnp.float32),
        scratch_shapes=[pltpu.VMEM((B, CB), jnp.float32)],
    )(x, mask)


# speedup vs baseline: 4.1621x; 1.0766x over previous
import jax
import jax.numpy as jnp
from jax.experimental import pallas as pl
from jax.experimental.pallas import tpu as pltpu

B, N = 128, 8192
CB = 2048
NBLK = N // CB


def _tc_body(x_ref, m_ref, o_ref, carry_ref):
    i = pl.program_id(0)

    @pl.when(i == 0)
    def _():
        carry_ref[...] = jnp.zeros_like(carry_ref)

    s = x_ref[...] * m_ref[...].astype(jnp.float32)
    sh = 1
    while sh < CB:
        s = s + jnp.pad(s[:, :-sh], ((0, 0), (sh, 0)))
        sh *= 2
    o_ref[...] = s + carry_ref[...]
    carry_ref[...] = carry_ref[...] + jnp.broadcast_to(s[:, CB - 1:CB], (B, CB))


def kernel(x, mask):
    return pl.pallas_call(
        _tc_body,
        grid=(NBLK,),
        in_specs=[
            pl.BlockSpec((B, CB), lambda i: (0, i)),
            pl.BlockSpec((B, CB), lambda i: (0, i)),
        ],
        out_specs=pl.BlockSpec((B, CB), lambda i: (0, i)),
        out_shape=jax.ShapeDtypeStruct((B, N), jnp.float32),
        scratch_shapes=[pltpu.VMEM((B, CB), jnp.float32)],
    )(x, mask)
